# Initial kernel scaffold; baseline (speedup 1.0000x reference)
import functools

import jax
import jax.numpy as jnp
from jax import lax
from jax.experimental import pallas as pl
from jax.experimental.pallas import tpu as pltpu
from jax.experimental.pallas import tpu_sc as plsc

_NC = 2
_NS = 16
_L = 16

_NPASS = 3
_NCHUNK = _NC * _NPASS
_K = 128
_CL = 16
_EB = 2000
_ZR = 64


def _sc_grid(n, r, e, d):
  s_total = r * n
  cr = pl.cdiv(s_total, _NCHUNK)
  cr = ((cr + _NS * 8 - 1) // (_NS * 8)) * (_NS * 8)
  share = cr // _NS
  ept = e // _NS
  nblk = ept // _EB
  cap = ept + _L
  return cr, share, ept, nblk, cap


def _sc_body(n, r, e, d, cr, share, ept, nblk, cap,
             x_hbm, adj_hbm, agg_hbm, cnt_hbm,
             esrc, edst, erel, src1d, seg1d, fsrc, fseg,
             rowsb, onesb, zrow, zcnt, agg_sp, cnt_sp):
  c = lax.axis_index("c")
  s = lax.axis_index("s")

  def _init_z(i, _):
    for j in range(d // _L):
      zrow[i, pl.ds(j * _L, _L)] = jnp.zeros((_L,), jnp.float32)
    zcnt[i, :] = jnp.zeros((_CL,), jnp.float32)
    return 0
  lax.fori_loop(0, _ZR, _init_z, 0)

  def _init_ones(i, _):
    onesb[i, :] = jnp.ones((_CL,), jnp.float32)
    return 0
  lax.fori_loop(0, _K, _init_ones, 0)

  tb = s * share
  ebase0 = s * ept
  nz = share // _ZR
  rz = share % _ZR

  def _pass_body(p, _):
    ci = c * _NPASS + p
    lo = ci * cr
    hi = lo + cr

    def _zloop(z, _):
      pltpu.sync_copy(zrow, agg_sp.at[pl.ds(tb + z * _ZR, _ZR)])
      pltpu.sync_copy(zcnt, cnt_sp.at[pl.ds(tb + z * _ZR, _ZR)])
      return 0
    lax.fori_loop(0, nz, _zloop, 0)
    if rz:
      pltpu.sync_copy(zrow.at[pl.ds(0, rz)], agg_sp.at[pl.ds(tb + nz * _ZR, rz)])
      pltpu.sync_copy(zcnt.at[pl.ds(0, rz)], cnt_sp.at[pl.ds(tb + nz * _ZR, rz)])
    plsc.subcore_barrier()

    iota = lax.iota(jnp.int32, _L)

    def _fire(bi, off):
      base = bi * _K
      for j in range(_K // _L):
        sl = pl.ds(base + j * _L, _L)
        valid = (base + j * _L + iota) < off
        fsrc[pl.ds(j * _L, _L)] = jnp.where(valid, src1d[sl], 0)
        fseg[pl.ds(j * _L, _L)] = jnp.where(valid, seg1d[sl], cr)
      pltpu.sync_copy(x_hbm.at[fsrc], rowsb)
      pltpu.sync_copy(rowsb, agg_sp.at[fseg], add=True)
      pltpu.sync_copy(onesb, cnt_sp.at[fseg], add=True)

    def _block_body(blk, st):
      off0, fired0 = st
      eb = ebase0 + blk * _EB
      pltpu.sync_copy(adj_hbm.at[0, pl.ds(eb, _EB)], esrc)
      pltpu.sync_copy(adj_hbm.at[1, pl.ds(eb, _EB)], edst)
      pltpu.sync_copy(adj_hbm.at[2, pl.ds(eb, _EB)], erel)

      def _vec_body(i, st2):
        off, fired = st2
        sl = pl.ds(i * _L, _L)
        sv = esrc[sl]
        dv = edst[sl]
        rv = erel[sl] % r
        seg = rv * n + dv
        m = (seg >= lo) & (seg < hi)
        plsc.store_compressed(src1d.at[pl.ds(off, _L)], sv, mask=m)
        plsc.store_compressed(seg1d.at[pl.ds(off, _L)], seg - lo, mask=m)
        off = off + jnp.sum(m.astype(jnp.int32))

        @pl.when(off >= (fired + 1) * _K)
        def _():
          _fire(fired, off)
        fired = jnp.where(off >= (fired + 1) * _K, fired + 1, fired)
        return off, fired

      return lax.fori_loop(0, _EB // _L, _vec_body, (off0, fired0))

    off, fired = lax.fori_loop(0, nblk, _block_body, (0, 0))

    @pl.when(off > fired * _K)
    def _():
      _fire(fired, off)

    plsc.subcore_barrier()
    pltpu.sync_copy(agg_sp.at[pl.ds(tb, share)], agg_hbm.at[pl.ds(lo + tb, share)])
    pltpu.sync_copy(cnt_sp.at[pl.ds(tb, share)], cnt_hbm.at[pl.ds(lo + tb, share)])
    plsc.subcore_barrier()
    return 0

  lax.fori_loop(0, _NPASS, _pass_body, 0)


def _tc_body(b, wc_ref, w_ref, agg_ref, cnt_ref, out_ref):
  rr = pl.program_id(1)
  wm = wc_ref[rr, 0] * w_ref[0]
  for j in range(1, b):
    wm = wm + wc_ref[rr, j] * w_ref[j]
  inv = 1.0 / jnp.maximum(cnt_ref[:, :1], 1.0)
  a = agg_ref[...] * inv
  contrib = jnp.dot(a, wm, preferred_element_type=jnp.float32)

  @pl.when(rr == 0)
  def _():
    out_ref[...] = contrib

  @pl.when(rr != 0)
  def _():
    out_ref[...] = out_ref[...] + contrib


def kernel(x, adj_t, weight, w_comp):
  n, d = x.shape
  e = adj_t.shape[1]
  b = weight.shape[0]
  r = w_comp.shape[0]
  d_out = weight.shape[2]

  cr, share, ept, nblk, cap = _sc_grid(n, r, e, d)
  srows = _NCHUNK * cr

  sc = pl.kernel(
      functools.partial(_sc_body, n, r, e, d, cr, share, ept, nblk, cap),
      out_type=[
          jax.ShapeDtypeStruct((srows, d), jnp.float32),
          jax.ShapeDtypeStruct((srows, _CL), jnp.float32),
      ],
      mesh=plsc.VectorSubcoreMesh(core_axis_name="c", subcore_axis_name="s"),
      scratch_types=[
          pltpu.VMEM((_EB,), jnp.int32),
          pltpu.VMEM((_EB,), jnp.int32),
          pltpu.VMEM((_EB,), jnp.int32),
          pltpu.VMEM((cap,), jnp.int32),
          pltpu.VMEM((cap,), jnp.int32),
          pltpu.VMEM((_K,), jnp.int32),
          pltpu.VMEM((_K,), jnp.int32),
          pltpu.VMEM((_K, d), jnp.float32),
          pltpu.VMEM((_K, _CL), jnp.float32),
          pltpu.VMEM((_ZR, d), jnp.float32),
          pltpu.VMEM((_ZR, _CL), jnp.float32),
          pltpu.VMEM_SHARED((cr + _L, d), jnp.float32),
          pltpu.VMEM_SHARED((cr + _L, _CL), jnp.float32),
      ],
  )
  agg, cnt = sc(x, adj_t)

  bn = 400
  nbn = n // bn
  out = pl.pallas_call(
      functools.partial(_tc_body, b),
      grid=(nbn, r),
      in_specs=[
          pl.BlockSpec(memory_space=pltpu.SMEM),
          pl.BlockSpec((b, d, d_out), lambda i, j: (0, 0, 0)),
          pl.BlockSpec((bn, d), lambda i, j: (j * nbn + i, 0)),
          pl.BlockSpec((bn, _CL), lambda i, j: (j * nbn + i, 0)),
      ],
      out_specs=pl.BlockSpec((bn, d_out), lambda i, j: (i, 0)),
      out_shape=jax.ShapeDtypeStruct((n, d_out), jnp.float32),
      compiler_params=pltpu.CompilerParams(
          dimension_semantics=("arbitrary", "arbitrary"),
      ),
  )(w_comp, weight, agg, cnt)
  return out


# speedup vs baseline: 5.9730x; 5.9730x over previous
import functools

import jax
import jax.numpy as jnp
from jax import lax
from jax.experimental import pallas as pl
from jax.experimental.pallas import tpu as pltpu
from jax.experimental.pallas import tpu_sc as plsc

_NC = 2
_NS = 16
_L = 16

_NPASS = 5
_NCHUNK = _NC * _NPASS
_K = 64
_CL = 16
_EB = 400
_ZR = 24


def _sc_grid(n, r, e, d):
  s_total = r * n
  cr = pl.cdiv(s_total, _NCHUNK)
  cr = ((cr + _NS * 8 - 1) // (_NS * 8)) * (_NS * 8)
  share = cr // _NS
  ept = e // _NS
  nblk = ept // _EB
  cap = ept + _L
  sb = cr.bit_length()
  assert (n << sb) < 2**31
  return cr, share, ept, nblk, cap, sb


def _sc_body(n, r, e, d, cr, share, ept, nblk, cap, sb,
             x_hbm, src_hbm, dst_hbm, rel_hbm, agg_hbm, cnt_hbm,
             esrc, edst, erel, pack1d, fsrc, fseg,
             rowsb, ones1, zrow, zc1, agg_sp, cnt_sp):
  c = lax.axis_index("c")
  s = lax.axis_index("s")
  zlen = ((share + _L - 1) // _L) * _L

  def _init_ones(i, _):
    ones1[pl.ds(i * _L, _L)] = jnp.ones((_L,), jnp.float32)
    return 0
  lax.fori_loop(0, _K // _L, _init_ones, 0)

  tb = s * share
  ebase0 = s * ept
  nz = share // _ZR
  rz = share % _ZR

  def _pass_body(p, _):
    ci = c * _NPASS + p
    lo = ci * cr
    hi = lo + cr

    def _init_z(i, _):
      for j in range(d // _L):
        zrow[i, pl.ds(j * _L, _L)] = jnp.zeros((_L,), jnp.float32)
      return 0
    lax.fori_loop(0, _ZR, _init_z, 0)

    def _init_zc(i, _):
      zc1[pl.ds(i * _L, _L)] = jnp.zeros((_L,), jnp.float32)
      return 0
    lax.fori_loop(0, zlen // _L, _init_zc, 0)

    def _zloop(z, _):
      pltpu.sync_copy(zrow, agg_sp.at[pl.ds(tb + z * _ZR, _ZR)])
      return 0
    lax.fori_loop(0, nz, _zloop, 0)
    if rz:
      pltpu.sync_copy(zrow.at[pl.ds(0, rz)], agg_sp.at[pl.ds(tb + nz * _ZR, rz)])
    pltpu.sync_copy(zc1.at[pl.ds(0, share)], cnt_sp.at[pl.ds(tb, share)])
    plsc.subcore_barrier()

    iota = lax.iota(jnp.int32, _L)

    zero_v = jnp.full((_L,), 0, jnp.int32)
    sac_v = jnp.full((_L,), cr, jnp.int32)

    def _fire(bi, off):
      base = bi * _K
      for j in range(_K // _L):
        sl = pl.ds(base + j * _L, _L)
        valid = (base + j * _L + iota) < off
        p = pack1d[sl]
        fsrc[pl.ds(j * _L, _L)] = jnp.where(valid, p >> sb, zero_v)
        fseg[pl.ds(j * _L, _L)] = jnp.where(valid, p & ((1 << sb) - 1), sac_v)
      pltpu.sync_copy(x_hbm.at[fsrc], rowsb)
      pltpu.sync_copy(rowsb, agg_sp.at[fseg], add=True)
      pltpu.sync_copy(ones1, cnt_sp.at[fseg], add=True)

    def _block_body(blk, st):
      off0, fired0 = st
      eb = ebase0 + blk * _EB
      pltpu.sync_copy(src_hbm.at[pl.ds(eb, _EB)], esrc)
      pltpu.sync_copy(dst_hbm.at[pl.ds(eb, _EB)], edst)
      pltpu.sync_copy(rel_hbm.at[pl.ds(eb, _EB)], erel)

      def _vec_body(i, st2):
        off, fired = st2
        sl = pl.ds(i * _L, _L)
        sv = esrc[sl]
        dv = edst[sl]
        rv = erel[sl]
        seg = (rv % r) * n + dv
        m = (seg >= lo) & (seg < hi)
        key = jnp.where(m, jnp.full((_L,), 0, jnp.int32),
                        jnp.full((_L,), 1, jnp.int32))
        packed = (sv << sb) | (seg - lo)
        _, pv = plsc.sort_key_val(key, packed)
        pack1d[pl.ds(off, _L)] = pv
        pc = plsc.all_reduce_population_count(m)
        off = off + pc[0]

        @pl.when(off >= (fired + 1) * _K)
        def _():
          _fire(fired, off)
        fired = jnp.where(off >= (fired + 1) * _K, fired + 1, fired)
        return off, fired

      return lax.fori_loop(0, _EB // _L, _vec_body, (off0, fired0))

    off, fired = lax.fori_loop(0, nblk, _block_body, (0, 0))

    @pl.when(off > fired * _K)
    def _():
      _fire(fired, off)

    plsc.subcore_barrier()

    def _eloop(q, _):
      pltpu.sync_copy(agg_sp.at[pl.ds(tb + q * _ZR, _ZR)], zrow)
      pltpu.sync_copy(zrow, agg_hbm.at[pl.ds(lo + tb + q * _ZR, _ZR)])
      return 0
    ne = share // _ZR
    re_ = share % _ZR
    lax.fori_loop(0, ne, _eloop, 0)
    if re_:
      qb = tb + ne * _ZR
      pltpu.sync_copy(agg_sp.at[pl.ds(qb, re_)], zrow.at[pl.ds(0, re_)])
      pltpu.sync_copy(zrow.at[pl.ds(0, re_)], agg_hbm.at[pl.ds(lo + qb, re_)])
    pltpu.sync_copy(cnt_sp.at[pl.ds(tb, share)], zc1.at[pl.ds(0, share)])
    pltpu.sync_copy(zc1.at[pl.ds(0, share)], cnt_hbm.at[pl.ds(lo + tb, share)])
    plsc.subcore_barrier()
    return 0

  lax.fori_loop(0, _NPASS, _pass_body, 0)


def _wm_body(r, b, wc_ref, wv_ref, t_ref):
  for rr in range(r):
    acc = wc_ref[rr, 0] * wv_ref[:, 0, :]
    for j in range(1, b):
      acc = acc + wc_ref[rr, j] * wv_ref[:, j, :]
    t_ref[:, rr, :] = acc


def _tc_body(wm_ref, agg_ref, cnt_ref, out_ref):
  rr = pl.program_id(1)
  wm = wm_ref[0]
  inv = 1.0 / jnp.maximum(cnt_ref[:, :1], 1.0)
  a = agg_ref[...] * inv
  contrib = jnp.dot(a, wm, preferred_element_type=jnp.float32)

  @pl.when(rr == 0)
  def _():
    out_ref[...] = contrib

  @pl.when(rr != 0)
  def _():
    out_ref[...] = out_ref[...] + contrib


def kernel(x, adj_t, weight, w_comp):
  n, d = x.shape
  e = adj_t.shape[1]
  b = weight.shape[0]
  r = w_comp.shape[0]
  d_out = weight.shape[2]

  cr, share, ept, nblk, cap, sb = _sc_grid(n, r, e, d)
  srows = _NCHUNK * cr

  sc = pl.kernel(
      functools.partial(_sc_body, n, r, e, d, cr, share, ept, nblk, cap, sb),
      out_type=[
          jax.ShapeDtypeStruct((srows, d), jnp.float32),
          jax.ShapeDtypeStruct((srows,), jnp.float32),
      ],
      mesh=plsc.VectorSubcoreMesh(core_axis_name="c", subcore_axis_name="s"),
      compiler_params=pltpu.CompilerParams(needs_layout_passes=False),
      scratch_types=[
          pltpu.VMEM((_EB,), jnp.int32),
          pltpu.VMEM((_EB,), jnp.int32),
          pltpu.VMEM((_EB,), jnp.int32),
          pltpu.VMEM((cap,), jnp.int32),
          pltpu.VMEM((_K,), jnp.int32),
          pltpu.VMEM((_K,), jnp.int32),
          pltpu.VMEM((_K, d), jnp.float32),
          pltpu.VMEM((_K,), jnp.float32),
          pltpu.VMEM((_ZR, d), jnp.float32),
          pltpu.VMEM((((share + _L - 1) // _L) * _L,), jnp.float32),
          pltpu.VMEM_SHARED((cr + _L, d), jnp.float32),
          pltpu.VMEM_SHARED((cr + _L,), jnp.float32),
      ],
  )
  agg, cnt = sc(x, adj_t[0], adj_t[1], adj_t[2])

  wv = weight.reshape(d, b, d_out)
  t = pl.pallas_call(
      functools.partial(_wm_body, r, b),
      in_specs=[
          pl.BlockSpec(memory_space=pltpu.SMEM),
          pl.BlockSpec((d, b, d_out), lambda: (0, 0, 0)),
      ],
      out_specs=pl.BlockSpec((d, r, d_out), lambda: (0, 0, 0)),
      out_shape=jax.ShapeDtypeStruct((d, r, d_out), jnp.float32),
  )(w_comp, wv)
  wmat = t.reshape(r, d, d_out)

  bn = 400
  nbn = n // bn
  out = pl.pallas_call(
      _tc_body,
      grid=(nbn, r),
      in_specs=[
          pl.BlockSpec((1, d, d_out), lambda i, j: (j, 0, 0)),
          pl.BlockSpec((bn, d), lambda i, j: (j * nbn + i, 0)),
          pl.BlockSpec((bn, 1), lambda i, j: (j * nbn + i, 0)),
      ],
      out_specs=pl.BlockSpec((bn, d_out), lambda i, j: (i, 0)),
      out_shape=jax.ShapeDtypeStruct((n, d_out), jnp.float32),
      compiler_params=pltpu.CompilerParams(
          dimension_semantics=("arbitrary", "arbitrary"),
      ),
  )(wmat, agg, cnt.reshape(srows, 1))
  return out


# speedup vs baseline: 7.9736x; 1.3349x over previous
import functools

import jax
import jax.numpy as jnp
from jax import lax
from jax.experimental import pallas as pl
from jax.experimental.pallas import tpu as pltpu
from jax.experimental.pallas import tpu_sc as plsc

_NC = 2
_NS = 16
_L = 16

_NPASS = 5
_NCHUNK = _NC * _NPASS
_K = 64
_CL = 16
_EB = 2000
_ZR = 24


def _sc_grid(n, r, e, d):
  s_total = r * n
  cr = pl.cdiv(s_total, _NCHUNK)
  cr = ((cr + _NS * 8 - 1) // (_NS * 8)) * (_NS * 8)
  share = cr // _NS
  ept = e // _NS
  nblk = ept // _EB
  cap = ept + _L
  sb = cr.bit_length()
  assert (n << sb) < 2**31
  return cr, share, ept, nblk, cap, sb


def _sc_body(n, r, e, d, cr, share, ept, nblk, cap, sb,
             x_hbm, src_hbm, dst_hbm, rel_hbm, agg_hbm, cnt_hbm,
             esrc, edst, erel, pack1d, fsrc, fseg,
             rowsb, ones1, zrow, zc1, agg_sp, cnt_sp,
             es0, es1, es2, gsem, asem, csem):
  c = lax.axis_index("c")
  s = lax.axis_index("s")
  zlen = ((share + _L - 1) // _L) * _L

  def _init_ones(i, _):
    ones1[pl.ds(i * _L, _L)] = jnp.ones((_L,), jnp.float32)
    return 0
  lax.fori_loop(0, _K // _L, _init_ones, 0)

  tb = s * share
  ebase0 = s * ept
  nz = share // _ZR
  rz = share % _ZR

  def _pass_body(p, _):
    ci = c * _NPASS + p
    lo = ci * cr
    hi = lo + cr

    def _init_z(i, _):
      for j in range(d // _L):
        zrow[i, pl.ds(j * _L, _L)] = jnp.zeros((_L,), jnp.float32)
      return 0
    lax.fori_loop(0, _ZR, _init_z, 0)

    def _init_zc(i, _):
      zc1[pl.ds(i * _L, _L)] = jnp.zeros((_L,), jnp.float32)
      return 0
    lax.fori_loop(0, zlen // _L, _init_zc, 0)

    def _zloop(z, _):
      pltpu.sync_copy(zrow, agg_sp.at[pl.ds(tb + z * _ZR, _ZR)])
      return 0
    lax.fori_loop(0, nz, _zloop, 0)
    if rz:
      pltpu.sync_copy(zrow.at[pl.ds(0, rz)], agg_sp.at[pl.ds(tb + nz * _ZR, rz)])
    pltpu.sync_copy(zc1.at[pl.ds(0, share)], cnt_sp.at[pl.ds(tb, share)])
    plsc.subcore_barrier()

    iota = lax.iota(jnp.int32, _L)

    zero_v = jnp.full((_L,), 0, jnp.int32)
    sac_v = jnp.full((_L,), cr, jnp.int32)

    def _fire(bi, off):
      base = bi * _K
      for j in range(_K // _L):
        sl = pl.ds(base + j * _L, _L)
        valid = (base + j * _L + iota) < off
        p = pack1d[sl]
        fsrc[pl.ds(j * _L, _L)] = jnp.where(valid, p >> sb, zero_v)
        fseg[pl.ds(j * _L, _L)] = jnp.where(valid, p & ((1 << sb) - 1), sac_v)
      pltpu.async_copy(x_hbm.at[fsrc], rowsb, gsem).wait()
      da = pltpu.async_copy(rowsb, agg_sp.at[fseg], asem, add=True)
      dc = pltpu.async_copy(ones1, cnt_sp.at[fseg], csem, add=True)
      da.wait()
      dc.wait()

    def _block_body(blk, st):
      off0, fired0 = st
      eb = ebase0 + blk * _EB
      d0 = pltpu.async_copy(src_hbm.at[pl.ds(eb, _EB)], esrc, es0)
      d1 = pltpu.async_copy(dst_hbm.at[pl.ds(eb, _EB)], edst, es1)
      d2 = pltpu.async_copy(rel_hbm.at[pl.ds(eb, _EB)], erel, es2)
      d0.wait()
      d1.wait()
      d2.wait()

      def _vec_body(i, st2):
        off, fired = st2
        sl = pl.ds(i * _L, _L)
        sv = esrc[sl]
        dv = edst[sl]
        rv = erel[sl]
        seg = (rv % r) * n + dv
        m = (seg >= lo) & (seg < hi)
        key = jnp.where(m, jnp.full((_L,), 0, jnp.int32),
                        jnp.full((_L,), 1, jnp.int32))
        packed = (sv << sb) | (seg - lo)
        _, pv = plsc.sort_key_val(key, packed)
        pack1d[pl.ds(off, _L)] = pv
        pc = plsc.all_reduce_population_count(m)
        off = off + pc[0]

        @pl.when(off >= (fired + 1) * _K)
        def _():
          _fire(fired, off)
        fired = jnp.where(off >= (fired + 1) * _K, fired + 1, fired)
        return off, fired

      return lax.fori_loop(0, _EB // _L, _vec_body, (off0, fired0))

    off, fired = lax.fori_loop(0, nblk, _block_body, (0, 0))

    @pl.when(off > fired * _K)
    def _():
      _fire(fired, off)

    plsc.subcore_barrier()

    def _eloop(q, _):
      pltpu.sync_copy(agg_sp.at[pl.ds(tb + q * _ZR, _ZR)], zrow)
      pltpu.sync_copy(zrow, agg_hbm.at[pl.ds(lo + tb + q * _ZR, _ZR)])
      return 0
    ne = share // _ZR
    re_ = share % _ZR
    lax.fori_loop(0, ne, _eloop, 0)
    if re_:
      qb = tb + ne * _ZR
      pltpu.sync_copy(agg_sp.at[pl.ds(qb, re_)], zrow.at[pl.ds(0, re_)])
      pltpu.sync_copy(zrow.at[pl.ds(0, re_)], agg_hbm.at[pl.ds(lo + qb, re_)])
    pltpu.sync_copy(cnt_sp.at[pl.ds(tb, share)], zc1.at[pl.ds(0, share)])
    pltpu.sync_copy(zc1.at[pl.ds(0, share)], cnt_hbm.at[pl.ds(lo + tb, share)])
    plsc.subcore_barrier()
    return 0

  lax.fori_loop(0, _NPASS, _pass_body, 0)


def _wm_body(r, b, wc_ref, wv_ref, t_ref):
  for rr in range(r):
    acc = wc_ref[rr, 0] * wv_ref[:, 0, :]
    for j in range(1, b):
      acc = acc + wc_ref[rr, j] * wv_ref[:, j, :]
    t_ref[:, rr, :] = acc


def _tc_body(wm_ref, agg_ref, cnt_ref, out_ref):
  rr = pl.program_id(1)
  wm = wm_ref[0]
  inv = 1.0 / jnp.maximum(cnt_ref[:, :1], 1.0)
  a = agg_ref[...] * inv
  contrib = jnp.dot(a, wm, preferred_element_type=jnp.float32)

  @pl.when(rr == 0)
  def _():
    out_ref[...] = contrib

  @pl.when(rr != 0)
  def _():
    out_ref[...] = out_ref[...] + contrib


def kernel(x, adj_t, weight, w_comp):
  n, d = x.shape
  e = adj_t.shape[1]
  b = weight.shape[0]
  r = w_comp.shape[0]
  d_out = weight.shape[2]

  cr, share, ept, nblk, cap, sb = _sc_grid(n, r, e, d)
  srows = _NCHUNK * cr

  sc = pl.kernel(
      functools.partial(_sc_body, n, r, e, d, cr, share, ept, nblk, cap, sb),
      out_type=[
          jax.ShapeDtypeStruct((srows, d), jnp.float32),
          jax.ShapeDtypeStruct((srows,), jnp.float32),
      ],
      mesh=plsc.VectorSubcoreMesh(core_axis_name="c", subcore_axis_name="s"),
      compiler_params=pltpu.CompilerParams(needs_layout_passes=False),
      scratch_types=[
          pltpu.VMEM((_EB,), jnp.int32),
          pltpu.VMEM((_EB,), jnp.int32),
          pltpu.VMEM((_EB,), jnp.int32),
          pltpu.VMEM((cap,), jnp.int32),
          pltpu.VMEM((_K,), jnp.int32),
          pltpu.VMEM((_K,), jnp.int32),
          pltpu.VMEM((_K, d), jnp.float32),
          pltpu.VMEM((_K,), jnp.float32),
          pltpu.VMEM((_ZR, d), jnp.float32),
          pltpu.VMEM((((share + _L - 1) // _L) * _L,), jnp.float32),
          pltpu.VMEM_SHARED((cr + _L, d), jnp.float32),
          pltpu.VMEM_SHARED((cr + _L,), jnp.float32),
          pltpu.SemaphoreType.DMA,
          pltpu.SemaphoreType.DMA,
          pltpu.SemaphoreType.DMA,
          pltpu.SemaphoreType.DMA,
          pltpu.SemaphoreType.DMA,
          pltpu.SemaphoreType.DMA,
      ],
  )
  agg, cnt = sc(x, adj_t[0], adj_t[1], adj_t[2])

  wv = weight.reshape(d, b, d_out)
  t = pl.pallas_call(
      functools.partial(_wm_body, r, b),
      in_specs=[
          pl.BlockSpec(memory_space=pltpu.SMEM),
          pl.BlockSpec((d, b, d_out), lambda: (0, 0, 0)),
      ],
      out_specs=pl.BlockSpec((d, r, d_out), lambda: (0, 0, 0)),
      out_shape=jax.ShapeDtypeStruct((d, r, d_out), jnp.float32),
  )(w_comp, wv)
  wmat = t.reshape(r, d, d_out)

  bn = 400
  nbn = n // bn
  out = pl.pallas_call(
      _tc_body,
      grid=(nbn, r),
      in_specs=[
          pl.BlockSpec((1, d, d_out), lambda i, j: (j, 0, 0)),
          pl.BlockSpec((bn, d), lambda i, j: (j * nbn + i, 0)),
          pl.BlockSpec((bn, 1), lambda i, j: (j * nbn + i, 0)),
      ],
      out_specs=pl.BlockSpec((bn, d_out), lambda i, j: (i, 0)),
      out_shape=jax.ShapeDtypeStruct((n, d_out), jnp.float32),
      compiler_params=pltpu.CompilerParams(
          dimension_semantics=("arbitrary", "arbitrary"),
      ),
  )(wmat, agg, cnt.reshape(srows, 1))
  return out


# speedup vs baseline: 8.7710x; 1.1000x over previous
import functools

import jax
import jax.numpy as jnp
from jax import lax
from jax.experimental import pallas as pl
from jax.experimental.pallas import tpu as pltpu
from jax.experimental.pallas import tpu_sc as plsc

_NC = 2
_NS = 16
_L = 16

_NPASS = 5
_NCHUNK = _NC * _NPASS
_K = 64
_CL = 16
_EB = 2000
_ZR = 16
_CAP = 6016


def _sc_grid(n, r, e, d):
  s_total = r * n
  cr = pl.cdiv(s_total, _NCHUNK)
  cr = ((cr + _NS * 8 - 1) // (_NS * 8)) * (_NS * 8)
  share = cr // _NS
  ept = e // _NS
  nblk = ept // _EB
  cap = _CAP + 2 * _K
  sb = cr.bit_length()
  assert (n << sb) < 2**31
  return cr, share, ept, nblk, cap, sb


def _sc_body(n, r, e, d, cr, share, ept, nblk, cap, sb,
             x_hbm, src_hbm, dst_hbm, rel_hbm, agg_hbm, cnt_hbm,
             esrc, edst, erel, pack1d, fsrc2, fseg2,
             rows2, ones1, zrow, zc1, agg_sp, cnt_sp,
             es0, es1, es2, gs0, gs1, as0, as1, cs0, cs1):
  gsems = (gs0, gs1)
  asems = (as0, as1)
  csems = (cs0, cs1)
  c = lax.axis_index("c")
  s = lax.axis_index("s")
  zlen = ((share + _L - 1) // _L) * _L

  def _init_ones(i, _):
    ones1[pl.ds(i * _L, _L)] = jnp.ones((_L,), jnp.float32)
    return 0
  lax.fori_loop(0, _K // _L, _init_ones, 0)

  tb = s * share
  ebase0 = s * ept
  nz = share // _ZR
  rz = share % _ZR

  def _pass_body(p, _):
    ci = c * _NPASS + p
    lo = ci * cr
    hi = lo + cr

    def _init_z(i, _):
      for j in range(d // _L):
        zrow[i, pl.ds(j * _L, _L)] = jnp.zeros((_L,), jnp.float32)
      return 0
    lax.fori_loop(0, _ZR, _init_z, 0)

    def _init_zc(i, _):
      zc1[pl.ds(i * _L, _L)] = jnp.zeros((_L,), jnp.float32)
      return 0
    lax.fori_loop(0, zlen // _L, _init_zc, 0)

    def _zloop(z, _):
      pltpu.sync_copy(zrow, agg_sp.at[pl.ds(tb + z * _ZR, _ZR)])
      return 0
    lax.fori_loop(0, nz, _zloop, 0)
    if rz:
      pltpu.sync_copy(zrow.at[pl.ds(0, rz)], agg_sp.at[pl.ds(tb + nz * _ZR, rz)])
    pltpu.sync_copy(zc1.at[pl.ds(0, share)], cnt_sp.at[pl.ds(tb, share)])
    plsc.subcore_barrier()

    iota = lax.iota(jnp.int32, _L)

    zero_v = jnp.full((_L,), 0, jnp.int32)
    sac_v = jnp.full((_L,), cr, jnp.int32)

    def _drain(off):
      nb = (off + _K - 1) // _K
      npair = (nb + 1) // 2

      def _pair(q, _):
        for sl in range(2):
          bi = 2 * q + sl

          @pl.when(q > 0)
          def _():
            pltpu.make_async_copy(
                rows2.at[sl], agg_sp.at[fseg2.at[sl]], asems[sl]).wait()
            pltpu.make_async_copy(
                ones1, cnt_sp.at[fseg2.at[sl]], csems[sl]).wait()
          base = bi * _K
          for j in range(_K // _L):
            sls = pl.ds(base + j * _L, _L)
            valid = (base + j * _L + iota) < off
            pv = pack1d[sls]
            fsrc2[sl, pl.ds(j * _L, _L)] = jnp.where(valid, pv >> sb, zero_v)
            fseg2[sl, pl.ds(j * _L, _L)] = jnp.where(
                valid, pv & ((1 << sb) - 1), sac_v)
          pltpu.async_copy(x_hbm.at[fsrc2.at[sl]], rows2.at[sl], gsems[sl])
        for sl in range(2):
          pltpu.make_async_copy(
              x_hbm.at[fsrc2.at[sl]], rows2.at[sl], gsems[sl]).wait()
          pltpu.async_copy(
              rows2.at[sl], agg_sp.at[fseg2.at[sl]], asems[sl], add=True)
          pltpu.async_copy(ones1, cnt_sp.at[fseg2.at[sl]], csems[sl], add=True)
        return 0

      lax.fori_loop(0, npair, _pair, 0)

      @pl.when(npair > 0)
      def _():
        for sl in range(2):
          pltpu.make_async_copy(
              rows2.at[sl], agg_sp.at[fseg2.at[sl]], asems[sl]).wait()
          pltpu.make_async_copy(
              ones1, cnt_sp.at[fseg2.at[sl]], csems[sl]).wait()

    def _block_body(blk, off0):
      eb = ebase0 + blk * _EB
      d0 = pltpu.async_copy(src_hbm.at[pl.ds(eb, _EB)], esrc, es0)
      d1 = pltpu.async_copy(dst_hbm.at[pl.ds(eb, _EB)], edst, es1)
      d2 = pltpu.async_copy(rel_hbm.at[pl.ds(eb, _EB)], erel, es2)
      d0.wait()
      d1.wait()
      d2.wait()

      def _vec_body(i, off):
        sl = pl.ds(i * _L, _L)
        sv = esrc[sl]
        dv = edst[sl]
        rv = erel[sl]
        seg = (rv % r) * n + dv
        m = (seg >= lo) & (seg < hi)
        key = jnp.where(m, jnp.full((_L,), 0, jnp.int32),
                        jnp.full((_L,), 1, jnp.int32))
        packed = (sv << sb) | (seg - lo)
        _, pv = plsc.sort_key_val(key, packed)
        pack1d[pl.ds(off, _L)] = pv
        pc = plsc.all_reduce_population_count(m)
        return off + pc[0]

      off = lax.fori_loop(0, _EB // _L, _vec_body, off0)

      full = off >= _CAP - _EB

      @pl.when(full)
      def _():
        _drain(off)
      return jnp.where(full, 0, off)

    off = lax.fori_loop(0, nblk, _block_body, 0)
    _drain(off)

    plsc.subcore_barrier()

    def _eloop(q, _):
      pltpu.sync_copy(agg_sp.at[pl.ds(tb + q * _ZR, _ZR)], zrow)
      pltpu.sync_copy(zrow, agg_hbm.at[pl.ds(lo + tb + q * _ZR, _ZR)])
      return 0
    ne = share // _ZR
    re_ = share % _ZR
    lax.fori_loop(0, ne, _eloop, 0)
    if re_:
      qb = tb + ne * _ZR
      pltpu.sync_copy(agg_sp.at[pl.ds(qb, re_)], zrow.at[pl.ds(0, re_)])
      pltpu.sync_copy(zrow.at[pl.ds(0, re_)], agg_hbm.at[pl.ds(lo + qb, re_)])
    pltpu.sync_copy(cnt_sp.at[pl.ds(tb, share)], zc1.at[pl.ds(0, share)])
    pltpu.sync_copy(zc1.at[pl.ds(0, share)], cnt_hbm.at[pl.ds(lo + tb, share)])
    plsc.subcore_barrier()
    return 0

  lax.fori_loop(0, _NPASS, _pass_body, 0)


def _wm_body(r, b, wc_ref, wv_ref, t_ref):
  for rr in range(r):
    acc = wc_ref[rr, 0] * wv_ref[:, 0, :]
    for j in range(1, b):
      acc = acc + wc_ref[rr, j] * wv_ref[:, j, :]
    t_ref[:, rr, :] = acc


def _tc_body(wm_ref, agg_ref, cnt_ref, out_ref):
  rr = pl.program_id(1)
  wm = wm_ref[0]
  inv = 1.0 / jnp.maximum(cnt_ref[:, :1], 1.0)
  a = agg_ref[...] * inv
  contrib = jnp.dot(a, wm, preferred_element_type=jnp.float32)

  @pl.when(rr == 0)
  def _():
    out_ref[...] = contrib

  @pl.when(rr != 0)
  def _():
    out_ref[...] = out_ref[...] + contrib


def kernel(x, adj_t, weight, w_comp):
  n, d = x.shape
  e = adj_t.shape[1]
  b = weight.shape[0]
  r = w_comp.shape[0]
  d_out = weight.shape[2]

  cr, share, ept, nblk, cap, sb = _sc_grid(n, r, e, d)
  srows = _NCHUNK * cr

  sc = pl.kernel(
      functools.partial(_sc_body, n, r, e, d, cr, share, ept, nblk, cap, sb),
      out_type=[
          jax.ShapeDtypeStruct((srows, d), jnp.float32),
          jax.ShapeDtypeStruct((srows,), jnp.float32),
      ],
      mesh=plsc.VectorSubcoreMesh(core_axis_name="c", subcore_axis_name="s"),
      compiler_params=pltpu.CompilerParams(needs_layout_passes=False),
      scratch_types=[
          pltpu.VMEM((_EB,), jnp.int32),
          pltpu.VMEM((_EB,), jnp.int32),
          pltpu.VMEM((_EB,), jnp.int32),
          pltpu.VMEM((cap,), jnp.int32),
          pltpu.VMEM((2, _K), jnp.int32),
          pltpu.VMEM((2, _K), jnp.int32),
          pltpu.VMEM((2, _K, d), jnp.float32),
          pltpu.VMEM((_K,), jnp.float32),
          pltpu.VMEM((_ZR, d), jnp.float32),
          pltpu.VMEM((((share + _L - 1) // _L) * _L,), jnp.float32),
          pltpu.VMEM_SHARED((cr + _L, d), jnp.float32),
          pltpu.VMEM_SHARED((cr + _L,), jnp.float32),
          pltpu.SemaphoreType.DMA,
          pltpu.SemaphoreType.DMA,
          pltpu.SemaphoreType.DMA,
          pltpu.SemaphoreType.DMA,
          pltpu.SemaphoreType.DMA,
          pltpu.SemaphoreType.DMA,
          pltpu.SemaphoreType.DMA,
          pltpu.SemaphoreType.DMA,
          pltpu.SemaphoreType.DMA,
      ],
  )
  agg, cnt = sc(x, adj_t[0], adj_t[1], adj_t[2])

  wv = weight.reshape(d, b, d_out)
  t = pl.pallas_call(
      functools.partial(_wm_body, r, b),
      in_specs=[
          pl.BlockSpec(memory_space=pltpu.SMEM),
          pl.BlockSpec((d, b, d_out), lambda: (0, 0, 0)),
      ],
      out_specs=pl.BlockSpec((d, r, d_out), lambda: (0, 0, 0)),
      out_shape=jax.ShapeDtypeStruct((d, r, d_out), jnp.float32),
  )(w_comp, wv)
  wmat = t.reshape(r, d, d_out)

  bn = 400
  nbn = n // bn
  out = pl.pallas_call(
      _tc_body,
      grid=(nbn, r),
      in_specs=[
          pl.BlockSpec((1, d, d_out), lambda i, j: (j, 0, 0)),
          pl.BlockSpec((bn, d), lambda i, j: (j * nbn + i, 0)),
          pl.BlockSpec((bn, 1), lambda i, j: (j * nbn + i, 0)),
      ],
      out_specs=pl.BlockSpec((bn, d_out), lambda i, j: (i, 0)),
      out_shape=jax.ShapeDtypeStruct((n, d_out), jnp.float32),
      compiler_params=pltpu.CompilerParams(
          dimension_semantics=("arbitrary", "arbitrary"),
      ),
  )(wmat, agg, cnt.reshape(srows, 1))
  return out
